# baseline (speedup 1.0000x reference)
import functools

import jax
import jax.numpy as jnp
from jax import lax
from jax.experimental import pallas as pl
from jax.experimental.pallas import tpu as pltpu
from jax.experimental.pallas import tpu_sc as plsc

_K = 20
_P = 5
_INT_MIN = -2147483648
_NW = 32

_NET8 = [(0, 1), (2, 3), (4, 5), (6, 7),
         (0, 2), (1, 3), (4, 6), (5, 7),
         (1, 2), (5, 6),
         (0, 4), (1, 5), (2, 6), (3, 7),
         (2, 4), (3, 5),
         (1, 2), (3, 4)]
_MERGE5 = [(0, 4), (1, 3), (2, 4), (1, 2), (3, 4)]


def _topk_body(xrow_ref, xall_ref, idx_ref, m_ref):
    b = pl.program_id(0)
    xrow = xrow_ref[0]
    xall = xall_ref[0]
    t = xrow.shape[1]
    n = xall.shape[1]

    inner = jax.lax.dot_general(
        xrow, xall, (((0,), (0,)), ((), ())),
        preferred_element_type=jnp.float32)
    xx = jnp.sum(xall * xall, axis=0, keepdims=True)
    xxrow = jnp.sum(xrow * xrow, axis=0).reshape(t, 1)
    m = jnp.sum(xall, axis=0, keepdims=True) * (1.0 / xall.shape[0])
    score = (2.0 * inner - xxrow) - xx

    bits = jax.lax.bitcast_convert_type(score, jnp.int32)
    key = jnp.where(bits >= 0, bits, (~bits) ^ jnp.int32(_INT_MIN))

    g = n // 8
    iota = jax.lax.broadcasted_iota(jnp.int32, (t, g), 1)
    ks = []
    nc = []
    for j in range(8):
        ks.append(key[:, j * g:(j + 1) * g])
        nc.append((n - 1 - j * g) - iota)

    for i, j in _NET8:
        sw = ks[i] < ks[j]
        ks[i], ks[j] = (jnp.where(sw, ks[j], ks[i]),
                        jnp.where(sw, ks[i], ks[j]))
        nc[i], nc[j] = (jnp.where(sw, nc[j], nc[i]),
                        jnp.where(sw, nc[i], nc[j]))

    h = g // 2
    ck = []
    cn = []
    for i in range(_P):
        ak, an = ks[i][:, :h], nc[i][:, :h]
        bk, bn = ks[_P - 1 - i][:, h:], nc[_P - 1 - i][:, h:]
        sw = ak < bk
        ck.append(jnp.where(sw, bk, ak))
        cn.append(jnp.where(sw, bn, an))
    for i, j in _MERGE5:
        sw = ck[i] < ck[j]
        ck[i], ck[j] = (jnp.where(sw, ck[j], ck[i]),
                        jnp.where(sw, ck[i], ck[j]))
        cn[i], cn[j] = (jnp.where(sw, cn[j], cn[i]),
                        jnp.where(sw, cn[i], cn[j]))

    int_min = jnp.int32(_INT_MIN)
    idx_cols = []
    for _ in range(_K):
        kmax = jnp.max(ck[0], axis=1, keepdims=True)
        selv = ck[0] == kmax
        pk = jnp.where(selv, cn[0], int_min)
        cmax = jnp.max(pk, axis=1, keepdims=True)
        sel = selv & (pk == cmax)
        idx_cols.append((n - 1) - cmax)
        for q in range(_P - 1):
            ck[q] = jnp.where(sel, ck[q + 1], ck[q])
            cn[q] = jnp.where(sel, cn[q + 1], cn[q])
        ck[_P - 1] = jnp.where(sel, int_min, ck[_P - 1])
    idx = jnp.concatenate(idx_cols, axis=1) + b * n
    idx_ref[0] = idx
    m_ref[0] = m


def _sc_gather(m2, idxf):
    total = idxf.shape[0]
    rows = total // 128
    rows_per = rows // _NW
    idx2 = idxf.reshape(rows, 128)
    mesh = plsc.VectorSubcoreMesh(core_axis_name="c", subcore_axis_name="s",
                                  num_cores=2, num_subcores=16)

    @functools.partial(
        pl.kernel,
        out_type=jax.ShapeDtypeStruct((rows, 128), jnp.float32),
        mesh=mesh,
        scratch_types=[
            pltpu.VMEM((rows_per, 128), jnp.int32),
            pltpu.VMEM((rows_per, 128), jnp.float32),
            pltpu.SemaphoreType.DMA,
        ],
    )
    def k(m_hbm, idx_hbm, out_hbm, idx_v, mg_v, sem):
        wid = lax.axis_index("s") * 2 + lax.axis_index("c")
        base = wid * rows_per
        pltpu.sync_copy(idx_hbm.at[pl.ds(base, rows_per)], idx_v)
        copies = [
            pltpu.async_copy(m_hbm.at[idx_v.at[j]], mg_v.at[j], sem)
            for j in range(rows_per)
        ]
        for cp in copies:
            cp.wait()
        pltpu.sync_copy(mg_v, out_hbm.at[pl.ds(base, rows_per)])

    return k(m2, idx2).reshape(total)


def _expand_body(w_ref, mg_ref, out_ref):
    w = w_ref[...]
    out_ref[0] = w[:, :, None] * mg_ref[0][None]


def kernel(x, k, W):
    del k
    bsz, c, n = x.shape
    t = min(512, n)

    idx, m = pl.pallas_call(
        _topk_body,
        grid=(bsz, n // t),
        in_specs=[
            pl.BlockSpec((1, c, t), lambda b, r: (b, 0, r)),
            pl.BlockSpec((1, c, n), lambda b, r: (b, 0, 0)),
        ],
        out_specs=[
            pl.BlockSpec((1, t, _K), lambda b, r: (b, r, 0)),
            pl.BlockSpec((1, 1, n), lambda b, r: (b, 0, 0)),
        ],
        out_shape=[
            jax.ShapeDtypeStruct((bsz, n, _K), jnp.int32),
            jax.ShapeDtypeStruct((bsz, 1, n), jnp.float32),
        ],
    )(x, x)

    idxf = idx.reshape(-1)
    mgf = _sc_gather(m.reshape(bsz * n), idxf)

    w2 = W.reshape(-1, 1)
    mg4 = mgf.reshape(bsz, n, _K)
    t2 = min(512, n)
    out = pl.pallas_call(
        _expand_body,
        grid=(bsz, n // t2),
        in_specs=[
            pl.BlockSpec((w2.shape[0], 1), lambda b, r: (0, 0)),
            pl.BlockSpec((1, t2, _K), lambda b, r: (b, r, 0)),
        ],
        out_specs=pl.BlockSpec((1, w2.shape[0], t2, _K),
                               lambda b, r: (b, 0, r, 0)),
        out_shape=jax.ShapeDtypeStruct((bsz, w2.shape[0], n, _K),
                                       jnp.float32),
    )(w2, mg4)

    return out, idxf


# speedup vs baseline: 1.3957x; 1.3957x over previous
import functools

import jax
import jax.numpy as jnp
from jax import lax
from jax.experimental import pallas as pl
from jax.experimental.pallas import tpu as pltpu
from jax.experimental.pallas import tpu_sc as plsc

_K = 20
_P = 5
_INT_MIN = -2147483648
_NW = 32

_NET8 = [(0, 1), (2, 3), (4, 5), (6, 7),
         (0, 2), (1, 3), (4, 6), (5, 7),
         (1, 2), (5, 6),
         (0, 4), (1, 5), (2, 6), (3, 7),
         (2, 4), (3, 5),
         (1, 2), (3, 4)]
_MERGE5 = [(0, 4), (1, 3), (2, 4), (1, 2), (3, 4)]


def _topk_body(xrow_ref, xall_ref, idx_ref, m_ref):
    b = pl.program_id(0)
    xrow = xrow_ref[0]
    xall = xall_ref[0]
    t = xrow.shape[1]
    n = xall.shape[1]

    inner = jax.lax.dot_general(
        xrow, xall, (((0,), (0,)), ((), ())),
        preferred_element_type=jnp.float32)
    xx = jnp.sum(xall * xall, axis=0, keepdims=True)
    xxrow = jnp.sum(xrow * xrow, axis=0).reshape(t, 1)
    m = jnp.sum(xall, axis=0, keepdims=True) * (1.0 / xall.shape[0])
    score = (2.0 * inner - xxrow) - xx

    bits = jax.lax.bitcast_convert_type(score, jnp.int32)
    key = jnp.where(bits >= 0, bits, (~bits) ^ jnp.int32(_INT_MIN))

    g = n // 8
    iota = jax.lax.broadcasted_iota(jnp.int32, (t, g), 1)
    ks = []
    nc = []
    for j in range(8):
        ks.append(key[:, j * g:(j + 1) * g])
        nc.append((n - 1 - j * g) - iota)

    for i, j in _NET8:
        sw = ks[i] < ks[j]
        ks[i], ks[j] = (jnp.where(sw, ks[j], ks[i]),
                        jnp.where(sw, ks[i], ks[j]))
        nc[i], nc[j] = (jnp.where(sw, nc[j], nc[i]),
                        jnp.where(sw, nc[i], nc[j]))

    h = g // 2
    ck = []
    cn = []
    for i in range(_P):
        ak, an = ks[i][:, :h], nc[i][:, :h]
        bk, bn = ks[_P - 1 - i][:, h:], nc[_P - 1 - i][:, h:]
        sw = ak < bk
        ck.append(jnp.where(sw, bk, ak))
        cn.append(jnp.where(sw, bn, an))
    for i, j in _MERGE5:
        sw = ck[i] < ck[j]
        ck[i], ck[j] = (jnp.where(sw, ck[j], ck[i]),
                        jnp.where(sw, ck[i], ck[j]))
        cn[i], cn[j] = (jnp.where(sw, cn[j], cn[i]),
                        jnp.where(sw, cn[i], cn[j]))

    int_min = jnp.int32(_INT_MIN)
    idx_cols = []
    for _ in range(_K):
        kmax = jnp.max(ck[0], axis=1, keepdims=True)
        selv = ck[0] == kmax
        pk = jnp.where(selv, cn[0], int_min)
        cmax = jnp.max(pk, axis=1, keepdims=True)
        sel = selv & (pk == cmax)
        idx_cols.append((n - 1) - cmax)
        for q in range(_P - 1):
            ck[q] = jnp.where(sel, ck[q + 1], ck[q])
            cn[q] = jnp.where(sel, cn[q + 1], cn[q])
        ck[_P - 1] = jnp.where(sel, int_min, ck[_P - 1])
    idx = jnp.concatenate(idx_cols, axis=1) + b * n
    idx_ref[0] = idx
    m_ref[0] = m


def _sc_gather(m2, idxf):
    total = idxf.shape[0]
    rows = total // 128
    rows_per = rows // _NW
    idx2 = idxf.reshape(rows, 128)
    mesh = plsc.VectorSubcoreMesh(core_axis_name="c", subcore_axis_name="s",
                                  num_cores=2, num_subcores=16)

    @functools.partial(
        pl.kernel,
        out_type=jax.ShapeDtypeStruct((rows, 128), jnp.float32),
        mesh=mesh,
        scratch_types=[
            pltpu.VMEM((rows_per, 128), jnp.int32),
            pltpu.VMEM((rows_per, 128), jnp.float32),
            pltpu.SemaphoreType.DMA,
        ],
    )
    def k(m_hbm, idx_hbm, out_hbm, idx_v, mg_v, sem):
        wid = lax.axis_index("s") * 2 + lax.axis_index("c")
        base = wid * rows_per
        pltpu.sync_copy(idx_hbm.at[pl.ds(base, rows_per)], idx_v)
        copies = [
            pltpu.async_copy(m_hbm.at[idx_v.at[j]], mg_v.at[j], sem)
            for j in range(rows_per)
        ]
        for cp in copies:
            cp.wait()
        pltpu.sync_copy(mg_v, out_hbm.at[pl.ds(base, rows_per)])

    return k(m2, idx2).reshape(total)


def _expand_body(w_ref, mg_ref, out_ref):
    out_ref[0] = w_ref[...] * mg_ref[0]


def kernel(x, k, W):
    del k
    bsz, c, n = x.shape
    t = min(512, n)

    idx, m = pl.pallas_call(
        _topk_body,
        grid=(bsz, n // t),
        in_specs=[
            pl.BlockSpec((1, c, t), lambda b, r: (b, 0, r)),
            pl.BlockSpec((1, c, n), lambda b, r: (b, 0, 0)),
        ],
        out_specs=[
            pl.BlockSpec((1, t, _K), lambda b, r: (b, r, 0)),
            pl.BlockSpec((1, 1, n), lambda b, r: (b, 0, 0)),
        ],
        out_shape=[
            jax.ShapeDtypeStruct((bsz, n, _K), jnp.int32),
            jax.ShapeDtypeStruct((bsz, 1, n), jnp.float32),
        ],
    )(x, x)

    idxf = idx.reshape(-1)
    mgf = _sc_gather(m.reshape(bsz * n), idxf)

    nk = n * _K
    ch = min(5120, nk)
    w2 = W.reshape(-1, 1)
    mg_flat = mgf.reshape(bsz, 1, nk)
    out = pl.pallas_call(
        _expand_body,
        grid=(bsz, nk // ch),
        in_specs=[
            pl.BlockSpec((w2.shape[0], 1), lambda b, j: (0, 0)),
            pl.BlockSpec((1, 1, ch), lambda b, j: (b, 0, j)),
        ],
        out_specs=pl.BlockSpec((1, w2.shape[0], ch), lambda b, j: (b, 0, j)),
        out_shape=jax.ShapeDtypeStruct((bsz, w2.shape[0], nk), jnp.float32),
    )(w2, mg_flat)

    return out.reshape(bsz, w2.shape[0], n, _K), idxf
